# baseline (speedup 1.0000x reference)
import numpy as np
import jax
import jax.numpy as jnp
from jax.experimental import pallas as pl
from jax.experimental.pallas import tpu as pltpu

B = 4096
H = 1024
NT = 64
S = 512
T = 9
TM1 = T - 1

BB = 128
GRID = B // BB

_U32 = jnp.uint32
_TINY = np.float32(np.finfo(np.float32).tiny)


def _np_threefry_block(k1, k2, x0, x1):
    x0 = np.asarray(x0, np.uint32).copy()
    x1 = np.asarray(x1, np.uint32).copy()
    ks0, ks1 = np.uint32(k1), np.uint32(k2)
    ks2 = np.uint32(ks0 ^ ks1 ^ np.uint32(0x1BD11BDA))
    rot = [13, 15, 26, 6, 17, 29, 16, 24]
    x0 = x0 + ks0
    x1 = x1 + ks1
    ks = [ks0, ks1, ks2]
    for i in range(5):
        for r in (rot[:4] if i % 2 == 0 else rot[4:]):
            x0 = x0 + x1
            x1 = ((x1 << np.uint32(r)) | (x1 >> np.uint32(32 - r))).astype(np.uint32)
            x1 = x1 ^ x0
        x0 = x0 + ks[(i + 1) % 3]
        x1 = x1 + ks[(i + 2) % 3] + np.uint32(i + 1)
    return x0, x1


_s0, _s1 = _np_threefry_block(0, 42, np.zeros(3, np.uint32), np.arange(3, dtype=np.uint32))
_KS0 = (int(_s0[0]), int(_s1[0]))
_KS1 = (int(_s0[1]), int(_s1[1]))
_KS2 = (int(_s0[2]), int(_s1[2]))
_c0, _c1 = _np_threefry_block(_KS1[0], _KS1[1], np.zeros(2, np.uint32), np.arange(2, dtype=np.uint32))
_K1B = (int(_c0[1]), int(_c1[1]))

_ROT = (13, 15, 26, 6, 17, 29, 16, 24)


def _tf_rounds(key, x1):
    k1, k2 = key
    ks0 = _U32(k1)
    ks2 = _U32(k1 ^ k2 ^ 0x1BD11BDA)
    ksl = (ks0, _U32(k2), ks2)
    x0 = jnp.full_like(x1, ks0)
    for i in range(5):
        for r in (_ROT[:4] if i % 2 == 0 else _ROT[4:]):
            x0 = x0 + x1
            x1 = (x1 << _U32(r)) | (x1 >> _U32(32 - r))
            x1 = x1 ^ x0
        x0 = x0 + ksl[(i + 1) % 3]
        x1 = x1 + ksl[(i + 2) % 3] + _U32(i + 1)
    return x0 ^ x1


def _tf_bits(key, f):
    return _tf_rounds(key, f.astype(_U32) + _U32(key[1]))


def _neglogw(bits):
    mant = (bits >> _U32(9)) | _U32(0x3F800000)
    u = jax.lax.bitcast_convert_type(mant, jnp.float32) - jnp.float32(1.0)
    u = jnp.maximum(_TINY, u)
    return jnp.log(-jnp.log(u))


def _gumbel(key, f):
    return -_neglogw(_tf_bits(key, f))


def _main_kernel(q_ref, ope_ref, se_ref, nte_ref, poss_ref,
                 inds_ref, scales_ref, logps_ref, mcat_ref, ntv_ref):
    i = pl.program_id(0)
    b0 = i * BB

    @pl.when(i == 0)
    def _tables():
        q = q_ref[...]
        a = ope_ref[...] + q
        m = jax.lax.dot_general(a, se_ref[...], (((1,), (1,)), ((), ())))
        mmax = jnp.max(m, axis=1, keepdims=True)
        lse = mmax + jnp.log(jnp.sum(jnp.exp(m - mmax), axis=1, keepdims=True))
        mcat_ref[:, 0:S] = m
        mcat_ref[:, S:S + 1] = lse
        mcat_ref[:, S + 1:] = jnp.full((NT, 127), -1e30, jnp.float32)
        ntl = jax.lax.dot_general(q, nte_ref[...], (((1,), (1,)), ((), ())))
        lanes = jax.lax.broadcasted_iota(jnp.int32, (8, 16), 1)
        ntv_ref[...] = jnp.where(lanes < T,
                                 jnp.broadcast_to(jnp.pad(ntl, ((0, 0), (0, 7))), (8, 16)),
                                 -1e30)

    ntrow = ntv_ref[0:1, :]
    r16 = jax.lax.broadcasted_iota(jnp.int32, (BB, 16), 0) + b0
    j16 = jax.lax.broadcasted_iota(jnp.int32, (BB, 16), 1)
    g0 = _gumbel(_KS0, r16 * T + j16)
    v0 = ntrow + g0
    m0 = jnp.max(v0, axis=1, keepdims=True)
    idx = jnp.min(jnp.where(v0 == m0, j16, 16), axis=1, keepdims=True)
    poss16 = poss_ref[0:1, 0:16]
    snt = jnp.sum(jnp.where(j16 == idx, poss16, 0), axis=1, keepdims=True)
    lp_sel = jnp.max(jnp.where(j16 == idx, ntrow, -1e30), axis=1, keepdims=True)
    ntmax = jnp.max(ntrow, axis=1, keepdims=True)
    lse_nt = ntmax + jnp.log(jnp.sum(jnp.exp(ntrow - ntmax), axis=1, keepdims=True))
    acc = lp_sel - lse_nt

    r8 = jax.lax.broadcasted_iota(jnp.int32, (BB, TM1), 0) + b0
    t8 = jax.lax.broadcasted_iota(jnp.int32, (BB, TM1), 1)
    rbits = _tf_bits(_K1B, r8 * TM1 + t8)
    ind_raw = (rbits & _U32(63)).astype(jnp.int32)
    mask = t8 >= snt
    inds = jnp.where(mask, 0, ind_raw)
    inds_ref[...] = inds

    s_iota = jax.lax.broadcasted_iota(jnp.int32, (BB, S), 1)
    rrow = jax.lax.broadcasted_iota(jnp.int32, (BB, S), 0) + b0
    lane64 = jax.lax.broadcasted_iota(jnp.int32, (BB, NT), 1)
    x1_base = (rrow * (TM1 * S) + s_iota).astype(_U32) + _U32(_KS2[1])
    for t in range(TM1):
        ind_t = inds[:, t:t + 1]
        onehot = (lane64 == ind_t).astype(jnp.float32)
        gath = jax.lax.dot_general(onehot, mcat_ref[...], (((1,), (0,)), ((), ())),
                                   precision=jax.lax.Precision.HIGHEST)
        grow = gath[:, 0:S]
        glse = gath[:, S:S + 1]
        nlw = _neglogw(_tf_rounds(_KS2, x1_base + _U32(t * S)))
        v2 = grow - nlw
        m2 = jnp.max(v2, axis=1, keepdims=True)
        sstar = jnp.min(jnp.where(v2 == m2, s_iota, S), axis=1, keepdims=True)
        logit_at = jnp.max(jnp.where(s_iota == sstar, grow, -1e30), axis=1, keepdims=True)
        lp_t = jnp.where(mask[:, t:t + 1], 0.0, logit_at - glse)
        acc = acc + lp_t
        scales_ref[:, t:t + 1] = sstar

    logps_ref[...] = jnp.broadcast_to(acc, (BB, TM1))


def kernel(imgs, q_param, op_embs, num_transforms_embs, scale_embs, possible_num_sequential_transforms):
    del imgs
    q2 = q_param.reshape(1, H)
    poss = jnp.broadcast_to(
        jnp.pad(possible_num_sequential_transforms, (0, 119))[None, :], (8, 128))

    inds, scales, logps_w = pl.pallas_call(
        _main_kernel,
        grid=(GRID,),
        in_specs=[
            pl.BlockSpec((1, H), lambda i: (0, 0)),
            pl.BlockSpec((NT, H), lambda i: (0, 0)),
            pl.BlockSpec((S, H), lambda i: (0, 0)),
            pl.BlockSpec((T, H), lambda i: (0, 0)),
            pl.BlockSpec((8, 128), lambda i: (0, 0)),
        ],
        out_specs=[
            pl.BlockSpec((BB, TM1), lambda i: (i, 0)),
            pl.BlockSpec((BB, TM1), lambda i: (i, 0)),
            pl.BlockSpec((BB, TM1), lambda i: (i, 0)),
        ],
        out_shape=(
            jax.ShapeDtypeStruct((B, TM1), jnp.int32),
            jax.ShapeDtypeStruct((B, TM1), jnp.int32),
            jax.ShapeDtypeStruct((B, TM1), jnp.float32),
        ),
        scratch_shapes=[
            pltpu.VMEM((NT, S + 128), jnp.float32),
            pltpu.VMEM((8, 16), jnp.float32),
        ],
    )(q2, op_embs, scale_embs, num_transforms_embs, poss)

    return inds, scales, logps_w[:, 0]


# speedup vs baseline: 1.0642x; 1.0642x over previous
import numpy as np
import jax
import jax.numpy as jnp
from jax.experimental import pallas as pl
from jax.experimental.pallas import tpu as pltpu

B = 4096
H = 1024
NT = 64
S = 512
T = 9
TM1 = T - 1

BB = 256
GRID = B // BB

_U32 = jnp.uint32
_TINY = np.float32(np.finfo(np.float32).tiny)


def _np_threefry_block(k1, k2, x0, x1):
    x0 = np.asarray(x0, np.uint32).copy()
    x1 = np.asarray(x1, np.uint32).copy()
    ks0, ks1 = np.uint32(k1), np.uint32(k2)
    ks2 = np.uint32(ks0 ^ ks1 ^ np.uint32(0x1BD11BDA))
    rot = [13, 15, 26, 6, 17, 29, 16, 24]
    x0 = x0 + ks0
    x1 = x1 + ks1
    ks = [ks0, ks1, ks2]
    for i in range(5):
        for r in (rot[:4] if i % 2 == 0 else rot[4:]):
            x0 = x0 + x1
            x1 = ((x1 << np.uint32(r)) | (x1 >> np.uint32(32 - r))).astype(np.uint32)
            x1 = x1 ^ x0
        x0 = x0 + ks[(i + 1) % 3]
        x1 = x1 + ks[(i + 2) % 3] + np.uint32(i + 1)
    return x0, x1


_s0, _s1 = _np_threefry_block(0, 42, np.zeros(3, np.uint32), np.arange(3, dtype=np.uint32))
_KS0 = (int(_s0[0]), int(_s1[0]))
_KS1 = (int(_s0[1]), int(_s1[1]))
_KS2 = (int(_s0[2]), int(_s1[2]))
_c0, _c1 = _np_threefry_block(_KS1[0], _KS1[1], np.zeros(2, np.uint32), np.arange(2, dtype=np.uint32))
_K1B = (int(_c0[1]), int(_c1[1]))

_ROT = (13, 15, 26, 6, 17, 29, 16, 24)


def _tf_rounds(key, x1):
    k1, k2 = key
    ks0 = _U32(k1)
    ks2 = _U32(k1 ^ k2 ^ 0x1BD11BDA)
    ksl = (ks0, _U32(k2), ks2)
    x0 = jnp.full_like(x1, ks0)
    for i in range(5):
        for r in (_ROT[:4] if i % 2 == 0 else _ROT[4:]):
            x0 = x0 + x1
            x1 = (x1 << _U32(r)) | (x1 >> _U32(32 - r))
            x1 = x1 ^ x0
        x0 = x0 + ksl[(i + 1) % 3]
        x1 = x1 + ksl[(i + 2) % 3] + _U32(i + 1)
    return x0 ^ x1


def _tf_bits(key, f):
    return _tf_rounds(key, f.astype(_U32) + _U32(key[1]))


def _neglogw(bits):
    mant = (bits >> _U32(9)) | _U32(0x3F800000)
    u = jax.lax.bitcast_convert_type(mant, jnp.float32) - jnp.float32(1.0)
    u = jnp.maximum(_TINY, u)
    return jnp.log(-jnp.log(u))


def _gumbel(key, f):
    return -_neglogw(_tf_bits(key, f))


def _main_kernel(q_ref, ope_ref, se_ref, nte_ref, poss_ref,
                 inds_ref, scales_ref, logps_ref, mcat_ref, ntv_ref):
    i = pl.program_id(0)
    b0 = i * BB

    @pl.when(i == 0)
    def _tables():
        q = q_ref[...]
        a = ope_ref[...] + q
        m = jax.lax.dot_general(a, se_ref[...], (((1,), (1,)), ((), ())))
        mmax = jnp.max(m, axis=1, keepdims=True)
        lse = mmax + jnp.log(jnp.sum(jnp.exp(m - mmax), axis=1, keepdims=True))
        mcat_ref[:, 0:S] = m
        mcat_ref[:, S:S + 1] = lse
        mcat_ref[:, S + 1:] = jnp.full((NT, 127), -1e30, jnp.float32)
        ntl = jax.lax.dot_general(q, nte_ref[...], (((1,), (1,)), ((), ())))
        lanes = jax.lax.broadcasted_iota(jnp.int32, (8, 16), 1)
        ntv_ref[...] = jnp.where(lanes < T,
                                 jnp.broadcast_to(jnp.pad(ntl, ((0, 0), (0, 7))), (8, 16)),
                                 -1e30)

    ntrow = ntv_ref[0:1, :]
    r16 = jax.lax.broadcasted_iota(jnp.int32, (BB, 16), 0) + b0
    j16 = jax.lax.broadcasted_iota(jnp.int32, (BB, 16), 1)
    g0 = _gumbel(_KS0, r16 * T + j16)
    v0 = ntrow + g0
    m0 = jnp.max(v0, axis=1, keepdims=True)
    idx = jnp.min(jnp.where(v0 == m0, j16, 16), axis=1, keepdims=True)
    poss16 = poss_ref[0:1, 0:16]
    snt = jnp.sum(jnp.where(j16 == idx, poss16, 0), axis=1, keepdims=True)
    lp_sel = jnp.max(jnp.where(j16 == idx, ntrow, -1e30), axis=1, keepdims=True)
    ntmax = jnp.max(ntrow, axis=1, keepdims=True)
    lse_nt = ntmax + jnp.log(jnp.sum(jnp.exp(ntrow - ntmax), axis=1, keepdims=True))
    acc = lp_sel - lse_nt

    r8 = jax.lax.broadcasted_iota(jnp.int32, (BB, TM1), 0) + b0
    t8 = jax.lax.broadcasted_iota(jnp.int32, (BB, TM1), 1)
    rbits = _tf_bits(_K1B, r8 * TM1 + t8)
    ind_raw = (rbits & _U32(63)).astype(jnp.int32)
    mask = t8 >= snt
    inds = jnp.where(mask, 0, ind_raw)
    inds_ref[...] = inds

    s_iota = jax.lax.broadcasted_iota(jnp.int32, (BB, S), 1)
    rrow = jax.lax.broadcasted_iota(jnp.int32, (BB, S), 0) + b0
    lane64 = jax.lax.broadcasted_iota(jnp.int32, (BB, NT), 1)
    x1_base = (rrow * (TM1 * S) + s_iota).astype(_U32) + _U32(_KS2[1])
    for t in range(TM1):
        ind_t = inds[:, t:t + 1]
        onehot = (lane64 == ind_t).astype(jnp.float32)
        gath = jax.lax.dot_general(onehot, mcat_ref[...], (((1,), (0,)), ((), ())),
                                   precision=jax.lax.Precision.HIGHEST)
        grow = gath[:, 0:S]
        glse = gath[:, S:S + 1]
        nlw = _neglogw(_tf_rounds(_KS2, x1_base + _U32(t * S)))
        v2 = grow - nlw
        m2 = jnp.max(v2, axis=1, keepdims=True)
        sstar = jnp.min(jnp.where(v2 == m2, s_iota, S), axis=1, keepdims=True)
        logit_at = jnp.max(jnp.where(s_iota == sstar, grow, -1e30), axis=1, keepdims=True)
        lp_t = jnp.where(mask[:, t:t + 1], 0.0, logit_at - glse)
        acc = acc + lp_t
        scales_ref[:, t:t + 1] = sstar

    logps_ref[...] = jnp.broadcast_to(acc, (BB, TM1))


def kernel(imgs, q_param, op_embs, num_transforms_embs, scale_embs, possible_num_sequential_transforms):
    del imgs
    q2 = q_param.reshape(1, H)
    poss = jnp.broadcast_to(
        jnp.pad(possible_num_sequential_transforms, (0, 119))[None, :], (8, 128))

    inds, scales, logps_w = pl.pallas_call(
        _main_kernel,
        grid=(GRID,),
        in_specs=[
            pl.BlockSpec((1, H), lambda i: (0, 0)),
            pl.BlockSpec((NT, H), lambda i: (0, 0)),
            pl.BlockSpec((S, H), lambda i: (0, 0)),
            pl.BlockSpec((T, H), lambda i: (0, 0)),
            pl.BlockSpec((8, 128), lambda i: (0, 0)),
        ],
        out_specs=[
            pl.BlockSpec((BB, TM1), lambda i: (i, 0)),
            pl.BlockSpec((BB, TM1), lambda i: (i, 0)),
            pl.BlockSpec((BB, TM1), lambda i: (i, 0)),
        ],
        out_shape=(
            jax.ShapeDtypeStruct((B, TM1), jnp.int32),
            jax.ShapeDtypeStruct((B, TM1), jnp.int32),
            jax.ShapeDtypeStruct((B, TM1), jnp.float32),
        ),
        scratch_shapes=[
            pltpu.VMEM((NT, S + 128), jnp.float32),
            pltpu.VMEM((8, 16), jnp.float32),
        ],
    )(q2, op_embs, scale_embs, num_transforms_embs, poss)

    return inds, scales, logps_w[:, 0]
